# baseline (speedup 1.0000x reference)
import functools

import jax
import jax.numpy as jnp
from jax import lax
from jax.experimental import pallas as pl
from jax.experimental.pallas import tpu as pltpu
from jax.experimental.pallas import tpu_sc as plsc

VOCAB = 100000
MAX_LEN = 200
EMBED_DIM = 64
BATCH = 4096

_INFO = plsc.get_sparse_core_info()
NUM_CORES = _INFO.num_cores
NUM_SUBCORES = _INFO.num_subcores
NUM_WORKERS = NUM_CORES * NUM_SUBCORES

BCOLS = BATCH // NUM_WORKERS
LANES = 16
VECS_PER_ROW = EMBED_DIM // LANES
B_UNROLL = 4


def _body(xT_hbm, tok_hbm, pos_hbm, out_hbm,
          idx_v, pos_v, rows0, rows1, trans0, trans1,
          sem_g0, sem_g1, sem_s0, sem_s1):
    wid = lax.axis_index("s") * NUM_CORES + lax.axis_index("c")
    b0 = wid * BCOLS

    pltpu.sync_copy(xT_hbm.at[:, pl.ds(b0, BCOLS)], idx_v)
    pltpu.sync_copy(pos_hbm, pos_v)

    rows = (rows0, rows1)
    trans = (trans0, trans1)
    sems_g = (sem_g0, sem_g1)
    sems_s = (sem_s0, sem_s1)

    d_idx = [jnp.arange(LANES, dtype=jnp.int32) + k * LANES
             for k in range(VECS_PER_ROW)]

    def start_gather(l, p):
        pltpu.async_copy(tok_hbm.at[idx_v.at[l]], rows[p], sems_g[p])

    def wait_gather(p):
        pltpu.make_async_copy(tok_hbm.at[idx_v.at[0]], rows[p],
                              sems_g[p]).wait()

    def start_store(l, p):
        pltpu.async_copy(trans[p], out_hbm.at[l, :, pl.ds(b0, BCOLS)],
                         sems_s[p])

    def wait_store(p):
        pltpu.make_async_copy(trans[p], out_hbm.at[0, :, pl.ds(b0, BCOLS)],
                              sems_s[p]).wait()

    start_gather(0, 0)

    def do_l(l, p):
        wait_gather(p)

        @pl.when(l + 1 < MAX_LEN)
        def _():
            start_gather(l + 1, 1 - p)

        @pl.when(l >= 2)
        def _():
            wait_store(p)

        pos_vecs = tuple(pos_v[l, pl.ds(k * LANES, LANES)]
                         for k in range(VECS_PER_ROW))

        def b_body(i, carry):
            for u in range(B_UNROLL):
                b = i * B_UNROLL + u
                bvec = jnp.full((LANES,), 0, jnp.int32) + b
                for k in range(VECS_PER_ROW):
                    v = rows[p][b, pl.ds(k * LANES, LANES)] + carry[k]
                    plsc.store_scatter(trans[p], [d_idx[k], bvec], v)
            return carry

        lax.fori_loop(0, BCOLS // B_UNROLL, b_body, pos_vecs)
        start_store(l, p)

    def pair_body(g, carry):
        for q in range(2):
            do_l(2 * g + q, q)
        return carry

    lax.fori_loop(0, MAX_LEN // 2, pair_body, None)
    wait_store(0)
    wait_store(1)


def kernel(x, token_table, pos_table):
    xT = jnp.transpose(x).astype(jnp.int32)

    mesh = plsc.VectorSubcoreMesh(core_axis_name="c", subcore_axis_name="s")
    run = functools.partial(
        pl.kernel,
        out_type=jax.ShapeDtypeStruct((MAX_LEN, EMBED_DIM, BATCH),
                                      jnp.float32),
        mesh=mesh,
        scratch_types=[
            pltpu.VMEM((MAX_LEN, BCOLS), jnp.int32),
            pltpu.VMEM((MAX_LEN, EMBED_DIM), jnp.float32),
            pltpu.VMEM((BCOLS, EMBED_DIM), jnp.float32),
            pltpu.VMEM((BCOLS, EMBED_DIM), jnp.float32),
            pltpu.VMEM((EMBED_DIM, BCOLS), jnp.float32),
            pltpu.VMEM((EMBED_DIM, BCOLS), jnp.float32),
            pltpu.SemaphoreType.DMA,
            pltpu.SemaphoreType.DMA,
            pltpu.SemaphoreType.DMA,
            pltpu.SemaphoreType.DMA,
        ],
        compiler_params=pltpu.CompilerParams(use_tc_tiling_on_sc=False,
                                             needs_layout_passes=False),
    )(_body)

    out = run(xT, token_table, pos_table)
    return jnp.transpose(out, (2, 0, 1))


# speedup vs baseline: 1.6824x; 1.6824x over previous
import functools

import jax
import jax.numpy as jnp
from jax import lax
from jax.experimental import pallas as pl
from jax.experimental.pallas import tpu as pltpu
from jax.experimental.pallas import tpu_sc as plsc

VOCAB = 100000
MAX_LEN = 200
EMBED_DIM = 64
BATCH = 4096

_INFO = plsc.get_sparse_core_info()
NUM_CORES = _INFO.num_cores
NUM_SUBCORES = _INFO.num_subcores
NUM_WORKERS = NUM_CORES * NUM_SUBCORES

BCOLS = BATCH // NUM_WORKERS
LANES = 16
VECS_PER_ROW = EMBED_DIM // LANES
B_UNROLL = 4
TPITCH = BCOLS + 1


def _body(xT_hbm, tok_hbm, pos_hbm, out_hbm,
          idx_v, pos_v, rows0, rows1, trans0, trans1,
          sem_g0, sem_g1, sem_s0, sem_s1):
    wid = lax.axis_index("s") * NUM_CORES + lax.axis_index("c")
    b0 = wid * BCOLS

    pltpu.sync_copy(xT_hbm.at[:, pl.ds(b0, BCOLS)], idx_v)
    pltpu.sync_copy(pos_hbm, pos_v)

    rows = (rows0, rows1)
    trans = (trans0, trans1)
    sems_g = (sem_g0, sem_g1)
    sems_s = (sem_s0, sem_s1)

    d_idx = [jnp.arange(LANES, dtype=jnp.int32) + k * LANES
             for k in range(VECS_PER_ROW)]

    def start_gather(l, p):
        pltpu.async_copy(tok_hbm.at[idx_v.at[l]], rows[p], sems_g[p])

    def wait_gather(p):
        pltpu.make_async_copy(tok_hbm.at[idx_v.at[0]], rows[p],
                              sems_g[p]).wait()

    def start_store(l, p):
        pltpu.async_copy(trans[p].at[:, pl.ds(0, BCOLS)],
                         out_hbm.at[l, :, pl.ds(b0, BCOLS)], sems_s[p])

    def wait_store(p):
        pltpu.make_async_copy(trans[p].at[:, pl.ds(0, BCOLS)],
                              out_hbm.at[0, :, pl.ds(b0, BCOLS)],
                              sems_s[p]).wait()

    start_gather(0, 0)

    def do_l(l, p):
        wait_gather(p)

        @pl.when(l + 1 < MAX_LEN)
        def _():
            start_gather(l + 1, 1 - p)

        @pl.when(l >= 2)
        def _():
            wait_store(p)

        pos_vecs = tuple(pos_v[l, pl.ds(k * LANES, LANES)]
                         for k in range(VECS_PER_ROW))

        def b_body(i, carry):
            for u in range(B_UNROLL):
                b = i * B_UNROLL + u
                bvec = jnp.full((LANES,), 0, jnp.int32) + b
                for k in range(VECS_PER_ROW):
                    v = rows[p][b, pl.ds(k * LANES, LANES)] + carry[k]
                    plsc.store_scatter(trans[p], [d_idx[k], bvec], v)
            return carry

        lax.fori_loop(0, BCOLS // B_UNROLL, b_body, pos_vecs)
        start_store(l, p)

    def pair_body(g, carry):
        for q in range(2):
            do_l(2 * g + q, q)
        return carry

    lax.fori_loop(0, MAX_LEN // 2, pair_body, None)
    wait_store(0)
    wait_store(1)


def kernel(x, token_table, pos_table):
    xT = jnp.transpose(x).astype(jnp.int32)

    mesh = plsc.VectorSubcoreMesh(core_axis_name="c", subcore_axis_name="s")
    run = functools.partial(
        pl.kernel,
        out_type=jax.ShapeDtypeStruct((MAX_LEN, EMBED_DIM, BATCH),
                                      jnp.float32),
        mesh=mesh,
        scratch_types=[
            pltpu.VMEM((MAX_LEN, BCOLS), jnp.int32),
            pltpu.VMEM((MAX_LEN, EMBED_DIM), jnp.float32),
            pltpu.VMEM((BCOLS, EMBED_DIM), jnp.float32),
            pltpu.VMEM((BCOLS, EMBED_DIM), jnp.float32),
            pltpu.VMEM((EMBED_DIM, TPITCH), jnp.float32),
            pltpu.VMEM((EMBED_DIM, TPITCH), jnp.float32),
            pltpu.SemaphoreType.DMA,
            pltpu.SemaphoreType.DMA,
            pltpu.SemaphoreType.DMA,
            pltpu.SemaphoreType.DMA,
        ],
        compiler_params=pltpu.CompilerParams(use_tc_tiling_on_sc=False,
                                             needs_layout_passes=False),
    )(_body)

    out = run(xT, token_table, pos_table)
    return jnp.transpose(out, (2, 0, 1))


# speedup vs baseline: 2.6612x; 1.5818x over previous
import functools

import jax
import jax.numpy as jnp
from jax import lax
from jax.experimental import pallas as pl
from jax.experimental.pallas import tpu as pltpu
from jax.experimental.pallas import tpu_sc as plsc

VOCAB = 100000
MAX_LEN = 200
EMBED_DIM = 64
BATCH = 4096

_INFO = plsc.get_sparse_core_info()
NUM_CORES = _INFO.num_cores
NUM_SUBCORES = _INFO.num_subcores
NUM_WORKERS = NUM_CORES * NUM_SUBCORES

BCOLS = BATCH // NUM_WORKERS
LANES = 16
VECS_PER_ROW = EMBED_DIM // LANES
B_UNROLL = 4
TPITCH = BCOLS + 1


def _body(xT_hbm, tok_hbm, pos_hbm, out_hbm,
          idx_v, pos_v, rows0, rows1, trans0, trans1,
          sem_g0, sem_g1, sem_s0, sem_s1):
    wid = lax.axis_index("s") * NUM_CORES + lax.axis_index("c")
    b0 = wid * BCOLS

    pltpu.sync_copy(xT_hbm.at[:, pl.ds(b0, BCOLS)], idx_v)
    pltpu.sync_copy(pos_hbm, pos_v)

    rows = (rows0, rows1)
    trans = (trans0, trans1)
    sems_g = (sem_g0, sem_g1)
    sems_s = (sem_s0, sem_s1)

    d_idx = [jnp.arange(LANES, dtype=jnp.int32) + k * LANES
             for k in range(VECS_PER_ROW)]

    def start_gather(l, p):
        pltpu.async_copy(tok_hbm.at[idx_v.at[l]], rows[p], sems_g[p])

    def wait_gather(p):
        pltpu.make_async_copy(tok_hbm.at[idx_v.at[0]], rows[p],
                              sems_g[p]).wait()

    def start_store(l, p):
        pltpu.async_copy(trans[p].at[:, pl.ds(0, BCOLS)],
                         out_hbm.at[l, :, pl.ds(b0, BCOLS)], sems_s[p])

    def wait_store(p):
        pltpu.make_async_copy(trans[p].at[:, pl.ds(0, BCOLS)],
                              out_hbm.at[0, :, pl.ds(b0, BCOLS)],
                              sems_s[p]).wait()

    start_gather(0, 0)

    def do_l(l, p):
        wait_gather(p)

        @pl.when(l + 1 < MAX_LEN)
        def _():
            start_gather(l + 1, 1 - p)

        @pl.when(l >= 2)
        def _():
            wait_store(p)

        pos_vecs = tuple(pos_v[l, pl.ds(k * LANES, LANES)]
                         for k in range(VECS_PER_ROW))

        @functools.partial(plsc.parallel_loop, 0, BCOLS,
                           unroll=B_UNROLL, carry=pos_vecs)
        def b_body(b, pv):
            bvec = jnp.full((LANES,), 0, jnp.int32) + b
            for k in range(VECS_PER_ROW):
                v = rows[p][b, pl.ds(k * LANES, LANES)] + pv[k]
                plsc.store_scatter(trans[p], [d_idx[k], bvec], v)
            return pv
        start_store(l, p)

    def pair_body(g, carry):
        for q in range(2):
            do_l(2 * g + q, q)
        return carry

    lax.fori_loop(0, MAX_LEN // 2, pair_body, None)
    wait_store(0)
    wait_store(1)


def kernel(x, token_table, pos_table):
    xT = jnp.transpose(x).astype(jnp.int32)

    mesh = plsc.VectorSubcoreMesh(core_axis_name="c", subcore_axis_name="s")
    run = functools.partial(
        pl.kernel,
        out_type=jax.ShapeDtypeStruct((MAX_LEN, EMBED_DIM, BATCH),
                                      jnp.float32),
        mesh=mesh,
        scratch_types=[
            pltpu.VMEM((MAX_LEN, BCOLS), jnp.int32),
            pltpu.VMEM((MAX_LEN, EMBED_DIM), jnp.float32),
            pltpu.VMEM((BCOLS, EMBED_DIM), jnp.float32),
            pltpu.VMEM((BCOLS, EMBED_DIM), jnp.float32),
            pltpu.VMEM((EMBED_DIM, TPITCH), jnp.float32),
            pltpu.VMEM((EMBED_DIM, TPITCH), jnp.float32),
            pltpu.SemaphoreType.DMA,
            pltpu.SemaphoreType.DMA,
            pltpu.SemaphoreType.DMA,
            pltpu.SemaphoreType.DMA,
        ],
        compiler_params=pltpu.CompilerParams(use_tc_tiling_on_sc=False,
                                             needs_layout_passes=False),
    )(_body)

    out = run(xT, token_table, pos_table)
    return jnp.transpose(out, (2, 0, 1))


# speedup vs baseline: 2.6658x; 1.0017x over previous
import functools

import jax
import jax.numpy as jnp
from jax import lax
from jax.experimental import pallas as pl
from jax.experimental.pallas import tpu as pltpu
from jax.experimental.pallas import tpu_sc as plsc

VOCAB = 100000
MAX_LEN = 200
EMBED_DIM = 64
BATCH = 4096

_INFO = plsc.get_sparse_core_info()
NUM_CORES = _INFO.num_cores
NUM_SUBCORES = _INFO.num_subcores
NUM_WORKERS = NUM_CORES * NUM_SUBCORES

BCOLS = BATCH // NUM_WORKERS
LANES = 16
VECS_PER_ROW = EMBED_DIM // LANES
B_UNROLL = 4
TPITCH = BCOLS + 1


def _body(xT_hbm, tok_hbm, pos_hbm, out_hbm,
          idx_v, pos_v, rows0, rows1, trans0, trans1,
          sem_g0, sem_g1, sem_s0, sem_s1):
    wid = lax.axis_index("s") * NUM_CORES + lax.axis_index("c")
    b0 = wid * BCOLS

    pltpu.sync_copy(xT_hbm.at[:, pl.ds(b0, BCOLS)], idx_v)
    pltpu.sync_copy(pos_hbm, pos_v)

    rows = (rows0, rows1)
    trans = (trans0, trans1)
    sems_g = (sem_g0, sem_g1)
    sems_s = (sem_s0, sem_s1)

    d_idx = [jnp.arange(LANES, dtype=jnp.int32) + k * LANES
             for k in range(VECS_PER_ROW)]

    def start_gather(l, p):
        pltpu.async_copy(tok_hbm.at[idx_v.at[l]], rows[p], sems_g[p])

    def wait_gather(p):
        pltpu.make_async_copy(tok_hbm.at[idx_v.at[0]], rows[p],
                              sems_g[p]).wait()

    def start_store(l, p):
        pltpu.async_copy(trans[p].at[:, pl.ds(0, BCOLS)],
                         out_hbm.at[l, :, pl.ds(b0, BCOLS)], sems_s[p])

    def wait_store(p):
        pltpu.make_async_copy(trans[p].at[:, pl.ds(0, BCOLS)],
                              out_hbm.at[0, :, pl.ds(b0, BCOLS)],
                              sems_s[p]).wait()

    start_gather(0, 0)

    def do_l(l, p):
        wait_gather(p)

        @pl.when(l + 1 < MAX_LEN)
        def _():
            start_gather(l + 1, 1 - p)

        @pl.when(l >= 2)
        def _():
            wait_store(p)

        pos_vecs = tuple(pos_v[l, pl.ds(k * LANES, LANES)]
                         for k in range(VECS_PER_ROW))

        @functools.partial(plsc.parallel_loop, 0, BCOLS,
                           unroll=1, carry=pos_vecs)
        def b_body(b, pv):
            bvec = jnp.full((LANES,), 0, jnp.int32) + b
            for k in range(VECS_PER_ROW):
                v = rows[p][b, pl.ds(k * LANES, LANES)] + pv[k]
                plsc.store_scatter(trans[p], [d_idx[k], bvec], v)
            return pv
        start_store(l, p)

    def pair_body(g, carry):
        for q in range(2):
            do_l(2 * g + q, q)
        return carry

    lax.fori_loop(0, MAX_LEN // 2, pair_body, None)
    wait_store(0)
    wait_store(1)


def kernel(x, token_table, pos_table):
    xT = jnp.transpose(x).astype(jnp.int32)

    mesh = plsc.VectorSubcoreMesh(core_axis_name="c", subcore_axis_name="s")
    run = functools.partial(
        pl.kernel,
        out_type=jax.ShapeDtypeStruct((MAX_LEN, EMBED_DIM, BATCH),
                                      jnp.float32),
        mesh=mesh,
        scratch_types=[
            pltpu.VMEM((MAX_LEN, BCOLS), jnp.int32),
            pltpu.VMEM((MAX_LEN, EMBED_DIM), jnp.float32),
            pltpu.VMEM((BCOLS, EMBED_DIM), jnp.float32),
            pltpu.VMEM((BCOLS, EMBED_DIM), jnp.float32),
            pltpu.VMEM((EMBED_DIM, TPITCH), jnp.float32),
            pltpu.VMEM((EMBED_DIM, TPITCH), jnp.float32),
            pltpu.SemaphoreType.DMA,
            pltpu.SemaphoreType.DMA,
            pltpu.SemaphoreType.DMA,
            pltpu.SemaphoreType.DMA,
        ],
        compiler_params=pltpu.CompilerParams(use_tc_tiling_on_sc=False,
                                             needs_layout_passes=False),
    )(_body)

    out = run(xT, token_table, pos_table)
    return jnp.transpose(out, (2, 0, 1))


# speedup vs baseline: 4.4384x; 1.6650x over previous
import functools

import jax
import jax.numpy as jnp
from jax import lax
from jax.experimental import pallas as pl
from jax.experimental.pallas import tpu as pltpu
from jax.experimental.pallas import tpu_sc as plsc

VOCAB = 100000
MAX_LEN = 200
EMBED_DIM = 64
BATCH = 4096

_INFO = plsc.get_sparse_core_info()
NUM_CORES = _INFO.num_cores
NUM_SUBCORES = _INFO.num_subcores
NUM_WORKERS = NUM_CORES * NUM_SUBCORES

BCOLS = BATCH // NUM_WORKERS
LANES = 16
VECS_PER_ROW = EMBED_DIM // LANES
B_UNROLL = 4
TPITCH = BCOLS + 1


def _body(xT_hbm, tok_hbm, pos_hbm, out_hbm,
          idx_v, pos_v, rows0, rows1, trans0, trans1,
          sem_g0, sem_g1, sem_s0, sem_s1):
    wid = lax.axis_index("s") * NUM_CORES + lax.axis_index("c")
    b0 = wid * BCOLS

    pltpu.sync_copy(xT_hbm.at[:, pl.ds(b0, BCOLS)], idx_v)
    pltpu.sync_copy(pos_hbm, pos_v)

    rows = (rows0, rows1)
    trans = (trans0, trans1)
    sems_g = (sem_g0, sem_g1)
    sems_s = (sem_s0, sem_s1)

    d_idx = [jnp.arange(LANES, dtype=jnp.int32) + k * LANES
             for k in range(VECS_PER_ROW)]

    def start_gather(l, p):
        pltpu.async_copy(tok_hbm.at[idx_v.at[l]], rows[p], sems_g[p])

    def wait_gather(p):
        pltpu.make_async_copy(tok_hbm.at[idx_v.at[0]], rows[p],
                              sems_g[p]).wait()

    def start_store(l, p):
        for dt in range(EMBED_DIM // 8):
            pltpu.async_copy(
                trans[p].at[pl.ds(dt * 8, 8), pl.ds(0, BCOLS)],
                out_hbm.at[l * (EMBED_DIM // 8) * NUM_WORKERS
                           + dt * NUM_WORKERS + wid],
                sems_s[p])

    def wait_store(p):
        for dt in range(EMBED_DIM // 8):
            pltpu.make_async_copy(
                trans[p].at[pl.ds(dt * 8, 8), pl.ds(0, BCOLS)],
                out_hbm.at[dt], sems_s[p]).wait()

    start_gather(0, 0)

    def do_l(l, p):
        wait_gather(p)

        @pl.when(l + 1 < MAX_LEN)
        def _():
            start_gather(l + 1, 1 - p)

        @pl.when(l >= 2)
        def _():
            wait_store(p)

        pos_vecs = tuple(pos_v[l, pl.ds(k * LANES, LANES)]
                         for k in range(VECS_PER_ROW))

        @functools.partial(plsc.parallel_loop, 0, BCOLS,
                           unroll=1, carry=pos_vecs)
        def b_body(b, pv):
            bvec = jnp.full((LANES,), 0, jnp.int32) + b
            for k in range(VECS_PER_ROW):
                v = rows[p][b, pl.ds(k * LANES, LANES)] + pv[k]
                plsc.store_scatter(trans[p], [d_idx[k], bvec], v)
            return pv
        start_store(l, p)

    def pair_body(g, carry):
        for q in range(2):
            do_l(2 * g + q, q)
        return carry

    lax.fori_loop(0, MAX_LEN // 2, pair_body, None)
    wait_store(0)
    wait_store(1)


def kernel(x, token_table, pos_table):
    xT = jnp.transpose(x).astype(jnp.int32)

    mesh = plsc.VectorSubcoreMesh(core_axis_name="c", subcore_axis_name="s")
    run = functools.partial(
        pl.kernel,
        out_type=jax.ShapeDtypeStruct(
            (MAX_LEN * (EMBED_DIM // 8) * NUM_WORKERS, 8, BATCH // NUM_WORKERS),
            jnp.float32),
        mesh=mesh,
        scratch_types=[
            pltpu.VMEM((MAX_LEN, BCOLS), jnp.int32),
            pltpu.VMEM((MAX_LEN, EMBED_DIM), jnp.float32),
            pltpu.VMEM((BCOLS, EMBED_DIM), jnp.float32),
            pltpu.VMEM((BCOLS, EMBED_DIM), jnp.float32),
            pltpu.VMEM((EMBED_DIM, TPITCH), jnp.float32),
            pltpu.VMEM((EMBED_DIM, TPITCH), jnp.float32),
            pltpu.SemaphoreType.DMA,
            pltpu.SemaphoreType.DMA,
            pltpu.SemaphoreType.DMA,
            pltpu.SemaphoreType.DMA,
        ],
        compiler_params=pltpu.CompilerParams(use_tc_tiling_on_sc=False,
                                             needs_layout_passes=False),
    )(_body)

    out = run(xT, token_table, pos_table)
    out5 = out.reshape(MAX_LEN, EMBED_DIM // 8, NUM_WORKERS, 8, BCOLS)
    out5 = jnp.transpose(out5, (2, 4, 0, 1, 3))
    return out5.reshape(BATCH, MAX_LEN, EMBED_DIM)
